# baseline (speedup 1.0000x reference)
import functools

import jax
import jax.numpy as jnp
from jax import lax
from jax.experimental import pallas as pl
from jax.experimental.pallas import tpu as pltpu
from jax.experimental.pallas import tpu_sc as plsc


def _make_sc_gather(V, D, B, num_cores=None):
    info = plsc.get_sparse_core_info()
    NC, NS = info.num_cores, info.num_subcores
    if num_cores is not None:
        NC = num_cores
    NW = NC * NS
    b_per_w = B // NW
    assert B % NW == 0 and b_per_w % 128 == 0
    nchunk = b_per_w // 128
    mesh = plsc.VectorSubcoreMesh(core_axis_name="c", subcore_axis_name="s",
                                  num_cores=NC)

    nbuf = min(nchunk, 4)

    def gather_k(idx_hbm, table_hbm, out_hbm, idx_v, rows_v, *sems):
        gsems = sems[:nbuf]
        wsems = sems[nbuf:]
        wid = lax.axis_index("s") * NC + lax.axis_index("c")
        base = wid * b_per_w

        pltpu.sync_copy(idx_hbm.at[wid], idx_v)

        def gather_chunk(j):
            return pltpu.async_copy(
                table_hbm.at[idx_v.at[j]],
                rows_v.at[j % nbuf],
                gsems[j % nbuf],
            )

        gathers = [gather_chunk(j) for j in range(nbuf)]
        gathers += [None] * (nchunk - nbuf)
        writes = [None] * nchunk
        for j in range(nchunk):
            slot = j % nbuf
            gathers[j].wait()
            writes[j] = pltpu.async_copy(
                rows_v.at[slot],
                out_hbm.at[pl.ds(base + j * 128, 128)],
                wsems[slot],
            )
            k = j + nbuf
            if k < nchunk:
                writes[j].wait()
                gathers[k] = gather_chunk(k)
        for j in range(nchunk - nbuf, nchunk):
            writes[j].wait()

    gather_call = functools.partial(
        pl.kernel,
        mesh=mesh,
        out_type=jax.ShapeDtypeStruct((B, D), jnp.float32),
        scratch_types=[
            pltpu.VMEM((nchunk, 128), jnp.int32),
            pltpu.VMEM((nbuf, 128, D), jnp.float32),
        ] + [pltpu.SemaphoreType.DMA] * (2 * nbuf),
    )(gather_k)

    def run(item_indices, Item):
        idx3 = item_indices.reshape(NW, nchunk, 128)
        return gather_call(idx3, Item)

    return run


def _mlp_body(x_ref, u_ref, w1a_ref, w1b_ref, b1_ref, w2_ref, b2_ref,
              w3_ref, b3_ref, wo_ref, bo_ref, out_ref, h0_ref):
    f32, bf = jnp.float32, jnp.bfloat16

    @pl.when(pl.program_id(0) == 0)
    def _():
        u = u_ref[...]
        h0_ref[...] = (
            jnp.dot(u, w1a_ref[...], preferred_element_type=f32) + b1_ref[...])

    h0 = h0_ref[...]
    x = x_ref[...].astype(bf)
    h1 = jnp.maximum(jnp.dot(x, w1b_ref[...], preferred_element_type=f32) + h0, 0.0)
    h2 = jnp.maximum(
        jnp.dot(h1.astype(bf), w2_ref[...], preferred_element_type=f32) + b2_ref[...], 0.0)
    h3 = jnp.maximum(
        jnp.dot(h2.astype(bf), w3_ref[...], preferred_element_type=f32) + b3_ref[...], 0.0)
    logit = jnp.dot(h3.astype(bf), wo_ref[...], preferred_element_type=f32) + bo_ref[...]
    out_ref[...] = jax.nn.sigmoid(logit)


def _mlp(x, u, W1a, W1b, b1, W2, b2, W3, b3, Wo, bo, blk=2048, interpret=False):
    B, D = x.shape
    H1 = W1b.shape[1]
    H2 = W2.shape[1]
    H3 = W3.shape[1]
    grid = (B // blk,)
    full = lambda shape: pl.BlockSpec(shape, lambda i: (0, 0))
    return pl.pallas_call(
        _mlp_body,
        grid=grid,
        in_specs=[
            pl.BlockSpec((blk, D), lambda i: (i, 0)),
            full((1, D)),
            full((D, H1)),
            full((D, H1)),
            full((1, H1)),
            full((H1, H2)),
            full((1, H2)),
            full((H2, H3)),
            full((1, H3)),
            full((H3, 1)),
            full((1, 1)),
        ],
        out_specs=pl.BlockSpec((blk, 1), lambda i: (i, 0)),
        out_shape=jax.ShapeDtypeStruct((B, 1), jnp.float32),
        scratch_shapes=[pltpu.VMEM((1, H1), jnp.float32)],
        compiler_params=pltpu.CompilerParams(
            dimension_semantics=("arbitrary",),
        ),
        interpret=interpret,
    )(x, u, W1a, W1b, b1, W2, b2, W3, b3, Wo, bo)


def kernel(item_indices, Pu, Eu, Item, W1, b1, W2, b2, W3, b3, Wo, bo):
    B = item_indices.shape[0]
    V, D = Item.shape
    gather = _make_sc_gather(V, D, B)
    u = (Pu + Eu).reshape(1, D)
    W1a = W1[:D, :]
    W1b = W1[D:, :]
    x = gather(item_indices.astype(jnp.int32), Item)
    bf = jnp.bfloat16
    return _mlp(
        x, u, W1a, W1b.astype(bf),
        b1.reshape(1, -1), W2.astype(bf), b2.reshape(1, -1),
        W3.astype(bf), b3.reshape(1, -1), Wo.astype(bf), bo.reshape(1, 1),
    )


# speedup vs baseline: 1.0149x; 1.0149x over previous
import functools

import jax
import jax.numpy as jnp
from jax import lax
from jax.experimental import pallas as pl
from jax.experimental.pallas import tpu as pltpu
from jax.experimental.pallas import tpu_sc as plsc


def _make_sc_gather(V, D, B, num_cores=None):
    info = plsc.get_sparse_core_info()
    NC, NS = info.num_cores, info.num_subcores
    if num_cores is not None:
        NC = num_cores
    NW = NC * NS
    b_per_w = B // NW
    assert B % NW == 0 and b_per_w % 128 == 0
    nchunk = b_per_w // 128
    mesh = plsc.VectorSubcoreMesh(core_axis_name="c", subcore_axis_name="s",
                                  num_cores=NC)

    nbuf = min(nchunk, 4)

    def gather_k(idx_hbm, table_hbm, out_hbm, idx_v, rows_v, *sems):
        gsems = sems[:nbuf]
        wsems = sems[nbuf:]
        wid = lax.axis_index("s") * NC + lax.axis_index("c")
        base = wid * b_per_w

        pltpu.sync_copy(idx_hbm.at[wid], idx_v)

        def gather_chunk(j):
            return pltpu.async_copy(
                table_hbm.at[idx_v.at[j]],
                rows_v.at[j % nbuf],
                gsems[j % nbuf],
            )

        gathers = [gather_chunk(j) for j in range(nbuf)]
        gathers += [None] * (nchunk - nbuf)
        writes = [None] * nchunk
        for j in range(nchunk):
            slot = j % nbuf
            gathers[j].wait()
            writes[j] = pltpu.async_copy(
                rows_v.at[slot],
                out_hbm.at[pl.ds(base + j * 128, 128)],
                wsems[slot],
            )
            k = j + nbuf
            if k < nchunk:
                writes[j].wait()
                gathers[k] = gather_chunk(k)
        for j in range(nchunk - nbuf, nchunk):
            writes[j].wait()

    gather_call = functools.partial(
        pl.kernel,
        mesh=mesh,
        out_type=jax.ShapeDtypeStruct((B, D), jnp.float32),
        scratch_types=[
            pltpu.VMEM((nchunk, 128), jnp.int32),
            pltpu.VMEM((nbuf, 128, D), jnp.float32),
        ] + [pltpu.SemaphoreType.DMA] * (2 * nbuf),
    )(gather_k)

    def run(item_indices, Item):
        idx3 = item_indices.reshape(NW, nchunk, 128)
        return gather_call(idx3, Item)

    return run


def _mlp_body(x_ref, pu_ref, eu_ref, w1_ref, b1_ref, w2_ref, b2_ref,
              w3_ref, b3_ref, wo_ref, bo_ref, out_ref,
              h0_ref, w1b_ref, w2b_ref, w3b_ref, wob_ref):
    f32, bf = jnp.float32, jnp.bfloat16
    D = pu_ref.shape[1]

    @pl.when(pl.program_id(0) == 0)
    def _():
        u = pu_ref[...] + eu_ref[...]
        h0_ref[...] = (
            jnp.dot(u, w1_ref[0:D, :], preferred_element_type=f32)
            + b1_ref[...])
        w1b_ref[...] = w1_ref[D:, :].astype(bf)
        w2b_ref[...] = w2_ref[...].astype(bf)
        w3b_ref[...] = w3_ref[...].astype(bf)
        wob_ref[...] = wo_ref[...].astype(bf)

    h0 = h0_ref[...]
    x = x_ref[...].astype(bf)
    h1 = jnp.maximum(jnp.dot(x, w1b_ref[...], preferred_element_type=f32) + h0, 0.0)
    h2 = jnp.maximum(
        jnp.dot(h1.astype(bf), w2b_ref[...], preferred_element_type=f32) + b2_ref[...], 0.0)
    h3 = jnp.maximum(
        jnp.dot(h2.astype(bf), w3b_ref[...], preferred_element_type=f32) + b3_ref[...], 0.0)
    logit = jnp.dot(h3.astype(bf), wob_ref[...], preferred_element_type=f32) + bo_ref[...]
    out_ref[...] = jax.nn.sigmoid(logit)


def _mlp(x, Pu, Eu, W1, b1, W2, b2, W3, b3, Wo, bo, blk=2048, interpret=False):
    B, D = x.shape
    H1 = W1.shape[1]
    H2 = W2.shape[1]
    H3 = W3.shape[1]
    grid = (B // blk,)
    full = lambda shape: pl.BlockSpec(shape, lambda i: tuple(0 for _ in shape))
    return pl.pallas_call(
        _mlp_body,
        grid=grid,
        in_specs=[
            pl.BlockSpec((blk, D), lambda i: (i, 0)),
            full((1, D)),
            full((1, D)),
            full((2 * D, H1)),
            full((1, H1)),
            full((H1, H2)),
            full((1, H2)),
            full((H2, H3)),
            full((1, H3)),
            full((H3, 1)),
            full((1, 1)),
        ],
        out_specs=pl.BlockSpec((blk, 1), lambda i: (i, 0)),
        out_shape=jax.ShapeDtypeStruct((B, 1), jnp.float32),
        scratch_shapes=[
            pltpu.VMEM((1, H1), jnp.float32),
            pltpu.VMEM((D, H1), jnp.bfloat16),
            pltpu.VMEM((H1, H2), jnp.bfloat16),
            pltpu.VMEM((H2, H3), jnp.bfloat16),
            pltpu.VMEM((H3, 1), jnp.bfloat16),
        ],
        compiler_params=pltpu.CompilerParams(
            dimension_semantics=("arbitrary",),
        ),
        interpret=interpret,
    )(x, Pu, Eu, W1, b1, W2, b2, W3, b3, Wo, bo)


def kernel(item_indices, Pu, Eu, Item, W1, b1, W2, b2, W3, b3, Wo, bo):
    B = item_indices.shape[0]
    V, D = Item.shape
    gather = _make_sc_gather(V, D, B)
    x = gather(item_indices.astype(jnp.int32), Item)
    return _mlp(
        x, Pu, Eu, W1, b1.reshape(1, -1), W2, b2.reshape(1, -1),
        W3, b3.reshape(1, -1), Wo, bo.reshape(1, 1),
    )


# speedup vs baseline: 1.2039x; 1.1862x over previous
import functools

import jax
import jax.numpy as jnp
from jax import lax
from jax.experimental import pallas as pl
from jax.experimental.pallas import tpu as pltpu
from jax.experimental.pallas import tpu_sc as plsc


def _make_sc_gather(V, D, B, num_cores=None):
    info = plsc.get_sparse_core_info()
    NC, NS = info.num_cores, info.num_subcores
    if num_cores is not None:
        NC = num_cores
    NW = NC * NS
    b_per_w = B // NW
    assert B % NW == 0 and b_per_w % 128 == 0
    nchunk = b_per_w // 128
    mesh = plsc.VectorSubcoreMesh(core_axis_name="c", subcore_axis_name="s",
                                  num_cores=NC)

    nbuf = min(nchunk, 4)

    def gather_k(idx_hbm, table_hbm, out_hbm, idx_v, rows_v, *sems):
        gsems = sems[:nbuf]
        wsems = sems[nbuf:]
        wid = lax.axis_index("s") * NC + lax.axis_index("c")
        base = wid * b_per_w

        pltpu.sync_copy(idx_hbm.at[wid], idx_v)

        def gather_chunk(j):
            return pltpu.async_copy(
                table_hbm.at[idx_v.at[j]],
                rows_v.at[j % nbuf],
                gsems[j % nbuf],
            )

        gathers = [gather_chunk(j) for j in range(nbuf)]
        gathers += [None] * (nchunk - nbuf)
        writes = [None] * nchunk
        for j in range(nchunk):
            slot = j % nbuf
            gathers[j].wait()
            writes[j] = pltpu.async_copy(
                rows_v.at[slot],
                out_hbm.at[pl.ds(base + j * 128, 128)],
                wsems[slot],
            )
            k = j + nbuf
            if k < nchunk:
                writes[j].wait()
                gathers[k] = gather_chunk(k)
        for j in range(nchunk - nbuf, nchunk):
            writes[j].wait()

    gather_call = functools.partial(
        pl.kernel,
        mesh=mesh,
        out_type=jax.ShapeDtypeStruct((B, D), jnp.float32),
        scratch_types=[
            pltpu.VMEM((nchunk, 128), jnp.int32),
            pltpu.VMEM((nbuf, 128, D), jnp.float32),
        ] + [pltpu.SemaphoreType.DMA] * (2 * nbuf),
    )(gather_k)

    def run(item_indices, Item):
        idx3 = item_indices.reshape(NW, nchunk, 128)
        return gather_call(idx3, Item)

    return run


def _mlp_body(x_ref, pu_ref, eu_ref, w1_ref, b1_ref, w2_ref, b2_ref,
              w3_ref, b3_ref, wo_ref, bo_ref, out_ref,
              h0_ref, w1b_ref, w2b_ref, w3b_ref, wob_ref, b2b_ref, b3b_ref):
    f32, bf = jnp.float32, jnp.bfloat16
    D = pu_ref.shape[1]

    @pl.when(pl.program_id(0) == 0)
    def _():
        u = pu_ref[...] + eu_ref[...]
        h0_ref[...] = (
            jnp.dot(u, w1_ref[0:D, :], preferred_element_type=f32)
            + b1_ref[...]).astype(bf)
        w1b_ref[...] = w1_ref[D:, :].astype(bf)
        w2b_ref[...] = w2_ref[...].astype(bf)
        w3b_ref[...] = w3_ref[...].astype(bf)
        wob_ref[...] = wo_ref[...].astype(bf)
        b2b_ref[...] = b2_ref[...].astype(bf)
        b3b_ref[...] = b3_ref[...].astype(bf)

    h0 = h0_ref[...]
    x = x_ref[...].astype(bf)
    d1 = jnp.dot(x, w1b_ref[...], preferred_element_type=f32)
    h1 = jnp.maximum(d1.astype(bf) + h0, 0)
    d2 = jnp.dot(h1, w2b_ref[...], preferred_element_type=f32)
    h2 = jnp.maximum(d2.astype(bf) + b2b_ref[...], 0)
    d3 = jnp.dot(h2, w3b_ref[...], preferred_element_type=f32)
    h3 = jnp.maximum(d3.astype(bf) + b3b_ref[...], 0)
    logit_t = (jnp.dot(wob_ref[...], h3.T, preferred_element_type=f32)
               + bo_ref[...])
    out_ref[...] = jax.nn.sigmoid(logit_t)


def _mlp(x, Pu, Eu, W1, b1, W2, b2, W3, b3, Wo, bo, blk=2048, interpret=False):
    B, D = x.shape
    H1 = W1.shape[1]
    H2 = W2.shape[1]
    H3 = W3.shape[1]
    grid = (B // blk,)
    full = lambda shape: pl.BlockSpec(shape, lambda i: tuple(0 for _ in shape))
    return pl.pallas_call(
        _mlp_body,
        grid=grid,
        in_specs=[
            pl.BlockSpec((blk, D), lambda i: (i, 0)),
            full((1, D)),
            full((1, D)),
            full((2 * D, H1)),
            full((1, H1)),
            full((H1, H2)),
            full((1, H2)),
            full((H2, H3)),
            full((1, H3)),
            full((1, H3)),
            full((1, 1)),
        ],
        out_specs=pl.BlockSpec((1, blk), lambda i: (0, i)),
        out_shape=jax.ShapeDtypeStruct((1, B), jnp.float32),
        scratch_shapes=[
            pltpu.VMEM((1, H1), jnp.bfloat16),
            pltpu.VMEM((D, H1), jnp.bfloat16),
            pltpu.VMEM((H1, H2), jnp.bfloat16),
            pltpu.VMEM((H2, H3), jnp.bfloat16),
            pltpu.VMEM((1, H3), jnp.bfloat16),
            pltpu.VMEM((1, H2), jnp.bfloat16),
            pltpu.VMEM((1, H3), jnp.bfloat16),
        ],
        compiler_params=pltpu.CompilerParams(
            dimension_semantics=("arbitrary",),
        ),
        interpret=interpret,
    )(x, Pu, Eu, W1, b1, W2, b2, W3, b3, Wo, bo)


def kernel(item_indices, Pu, Eu, Item, W1, b1, W2, b2, W3, b3, Wo, bo):
    B = item_indices.shape[0]
    V, D = Item.shape
    gather = _make_sc_gather(V, D, B)
    x = gather(item_indices.astype(jnp.int32), Item)
    out = _mlp(
        x, Pu, Eu, W1, b1.reshape(1, -1), W2, b2.reshape(1, -1),
        W3, b3.reshape(1, -1), Wo.reshape(1, -1), bo.reshape(1, 1),
    )
    return out.reshape(B, 1)


# speedup vs baseline: 1.2453x; 1.0344x over previous
import functools

import jax
import jax.numpy as jnp
from jax import lax
from jax.experimental import pallas as pl
from jax.experimental.pallas import tpu as pltpu
from jax.experimental.pallas import tpu_sc as plsc


def _make_sc_gather(V, D, B, num_cores=None):
    info = plsc.get_sparse_core_info()
    NC, NS = info.num_cores, info.num_subcores
    if num_cores is not None:
        NC = num_cores
    NW = NC * NS
    b_per_w = B // NW
    assert B % NW == 0 and b_per_w % 128 == 0
    nchunk = b_per_w // 128
    mesh = plsc.VectorSubcoreMesh(core_axis_name="c", subcore_axis_name="s",
                                  num_cores=NC)

    nbuf = min(nchunk, 4)

    def gather_k(idx_hbm, table_hbm, out_hbm, idx_v, rows_v, *sems):
        gsems = sems[:nbuf]
        wsems = sems[nbuf:]
        wid = lax.axis_index("s") * NC + lax.axis_index("c")
        base = wid * b_per_w

        pltpu.sync_copy(idx_hbm.at[wid], idx_v)

        def gather_chunk(j):
            return pltpu.async_copy(
                table_hbm.at[idx_v.at[j]],
                rows_v.at[j % nbuf],
                gsems[j % nbuf],
            )

        gathers = [gather_chunk(j) for j in range(nbuf)]
        gathers += [None] * (nchunk - nbuf)
        writes = [None] * nchunk
        for j in range(nchunk):
            slot = j % nbuf
            gathers[j].wait()
            writes[j] = pltpu.async_copy(
                rows_v.at[slot],
                out_hbm.at[pl.ds(base + j * 128, 128)],
                wsems[slot],
            )
            k = j + nbuf
            if k < nchunk:
                writes[j].wait()
                gathers[k] = gather_chunk(k)
        for j in range(nchunk - nbuf, nchunk):
            writes[j].wait()

    gather_call = functools.partial(
        pl.kernel,
        mesh=mesh,
        out_type=jax.ShapeDtypeStruct((B, D), jnp.float32),
        scratch_types=[
            pltpu.VMEM((nchunk, 128), jnp.int32),
            pltpu.VMEM((nbuf, 128, D), jnp.float32),
        ] + [pltpu.SemaphoreType.DMA] * (2 * nbuf),
    )(gather_k)

    def run(item_indices, Item):
        idx3 = item_indices.reshape(NW, nchunk, 128)
        return gather_call(idx3, Item)

    return run


def _mlp_body(x_ref, pu_ref, eu_ref, w1_ref, b1_ref, w2_ref, b2_ref,
              w3_ref, b3_ref, wo_ref, bo_ref, out_ref,
              h0_ref, w1b_ref, w2b_ref, w3b_ref, wob_ref, b2b_ref, b3b_ref):
    f32, bf = jnp.float32, jnp.bfloat16
    D = pu_ref.shape[1]

    @pl.when(pl.program_id(0) == 0)
    def _():
        u = pu_ref[...] + eu_ref[...]
        h0_ref[...] = (
            jnp.dot(u, w1_ref[0:D, :], preferred_element_type=f32)
            + b1_ref[...]).astype(bf)
        w1b_ref[...] = w1_ref[D:, :].astype(bf)
        w2b_ref[...] = w2_ref[...].astype(bf)
        w3b_ref[...] = w3_ref[...].astype(bf)
        wob_ref[...] = wo_ref[...].astype(bf)
        b2b_ref[...] = b2_ref[...].astype(bf)
        b3b_ref[...] = b3_ref[...].astype(bf)

    h0 = h0_ref[...]
    x = x_ref[...].astype(bf)
    d1 = jnp.dot(x, w1b_ref[...], preferred_element_type=f32)
    h1 = jnp.maximum(d1.astype(bf) + h0, 0)
    d2 = jnp.dot(h1, w2b_ref[...], preferred_element_type=f32)
    h2 = jnp.maximum(d2.astype(bf) + b2b_ref[...], 0)
    d3 = jnp.dot(h2, w3b_ref[...], preferred_element_type=f32)
    h3 = jnp.maximum(d3.astype(bf) + b3b_ref[...], 0)
    logit_t = (jnp.dot(wob_ref[...], h3.T, preferred_element_type=f32)
               + bo_ref[...])
    out_ref[...] = jax.nn.sigmoid(logit_t)


def _mlp(x, Pu, Eu, W1, b1, W2, b2, W3, b3, Wo, bo, blk=4096, interpret=False):
    B, D = x.shape
    H1 = W1.shape[1]
    H2 = W2.shape[1]
    H3 = W3.shape[1]
    grid = (B // blk,)
    full = lambda shape: pl.BlockSpec(shape, lambda i: tuple(0 for _ in shape))
    return pl.pallas_call(
        _mlp_body,
        grid=grid,
        in_specs=[
            pl.BlockSpec((blk, D), lambda i: (i, 0)),
            full((1, D)),
            full((1, D)),
            full((2 * D, H1)),
            full((1, H1)),
            full((H1, H2)),
            full((1, H2)),
            full((H2, H3)),
            full((1, H3)),
            full((1, H3)),
            full((1, 1)),
        ],
        out_specs=pl.BlockSpec((1, blk), lambda i: (0, i)),
        out_shape=jax.ShapeDtypeStruct((1, B), jnp.float32),
        scratch_shapes=[
            pltpu.VMEM((1, H1), jnp.bfloat16),
            pltpu.VMEM((D, H1), jnp.bfloat16),
            pltpu.VMEM((H1, H2), jnp.bfloat16),
            pltpu.VMEM((H2, H3), jnp.bfloat16),
            pltpu.VMEM((1, H3), jnp.bfloat16),
            pltpu.VMEM((1, H2), jnp.bfloat16),
            pltpu.VMEM((1, H3), jnp.bfloat16),
        ],
        compiler_params=pltpu.CompilerParams(
            dimension_semantics=("arbitrary",),
        ),
        interpret=interpret,
    )(x, Pu, Eu, W1, b1, W2, b2, W3, b3, Wo, bo)


def kernel(item_indices, Pu, Eu, Item, W1, b1, W2, b2, W3, b3, Wo, bo):
    B = item_indices.shape[0]
    V, D = Item.shape
    gather = _make_sc_gather(V, D, B)
    x = gather(item_indices.astype(jnp.int32), Item)
    out = _mlp(
        x, Pu, Eu, W1, b1.reshape(1, -1), W2, b2.reshape(1, -1),
        W3, b3.reshape(1, -1), Wo.reshape(1, -1), bo.reshape(1, 1),
    )
    return out.reshape(B, 1)


# speedup vs baseline: 1.2464x; 1.0009x over previous
import functools

import jax
import jax.numpy as jnp
from jax import lax
from jax.experimental import pallas as pl
from jax.experimental.pallas import tpu as pltpu
from jax.experimental.pallas import tpu_sc as plsc


def _make_sc_gather(V, D, B, num_cores=None):
    info = plsc.get_sparse_core_info()
    NC, NS = info.num_cores, info.num_subcores
    if num_cores is not None:
        NC = num_cores
    NW = NC * NS
    b_per_w = B // NW
    assert B % NW == 0 and b_per_w % 128 == 0
    nchunk = b_per_w // 128
    mesh = plsc.VectorSubcoreMesh(core_axis_name="c", subcore_axis_name="s",
                                  num_cores=NC)

    nbuf = min(nchunk, 4)

    def gather_k(idx_hbm, table_hbm, out_hbm, idx_v, rows_v, *sems):
        gsems = sems[:nbuf]
        wsems = sems[nbuf:]
        wid = lax.axis_index("s") * NC + lax.axis_index("c")
        base = wid * b_per_w

        pltpu.sync_copy(idx_hbm.at[wid], idx_v)

        def gather_chunk(j):
            return pltpu.async_copy(
                table_hbm.at[idx_v.at[j]],
                rows_v.at[j % nbuf],
                gsems[j % nbuf],
            )

        gathers = [gather_chunk(j) for j in range(nbuf)]
        gathers += [None] * (nchunk - nbuf)
        writes = [None] * nchunk
        for j in range(nchunk):
            slot = j % nbuf
            gathers[j].wait()
            writes[j] = pltpu.async_copy(
                rows_v.at[slot],
                out_hbm.at[pl.ds(base + j * 128, 128)],
                wsems[slot],
            )
            k = j + nbuf
            if k < nchunk:
                writes[j].wait()
                gathers[k] = gather_chunk(k)
        for j in range(nchunk - nbuf, nchunk):
            writes[j].wait()

    gather_call = functools.partial(
        pl.kernel,
        mesh=mesh,
        out_type=jax.ShapeDtypeStruct((B, D), jnp.float32),
        scratch_types=[
            pltpu.VMEM((nchunk, 128), jnp.int32),
            pltpu.VMEM((nbuf, 128, D), jnp.float32),
        ] + [pltpu.SemaphoreType.DMA] * (2 * nbuf),
    )(gather_k)

    def run(item_indices, Item):
        idx3 = item_indices.reshape(NW, nchunk, 128)
        return gather_call(idx3, Item)

    return run


def _mlp_body(x_ref, pu_ref, eu_ref, w1_ref, b1_ref, w2_ref, b2_ref,
              w3_ref, b3_ref, wo_ref, bo_ref, out_ref,
              h0_ref, w1b_ref, w2b_ref, w3b_ref, wob_ref, b2b_ref, b3b_ref):
    f32, bf = jnp.float32, jnp.bfloat16
    D = pu_ref.shape[1]

    @pl.when(pl.program_id(0) == 0)
    def _():
        u = pu_ref[...] + eu_ref[...]
        h0_ref[...] = (
            jnp.dot(u, w1_ref[0:D, :], preferred_element_type=f32)
            + b1_ref[...]).astype(bf)
        w1b_ref[...] = w1_ref[D:, :].astype(bf)
        w2b_ref[...] = w2_ref[...].astype(bf)
        w3b_ref[...] = w3_ref[...].astype(bf)
        wob_ref[...] = wo_ref[...].astype(bf)
        b2b_ref[...] = b2_ref[...].astype(bf)
        b3b_ref[...] = b3_ref[...].astype(bf)

    h0 = h0_ref[...]
    x = x_ref[...].astype(bf)
    d1 = jnp.dot(x, w1b_ref[...], preferred_element_type=f32)
    h1 = jnp.maximum(d1.astype(bf) + h0, 0)
    d2 = jnp.dot(h1, w2b_ref[...], preferred_element_type=f32)
    h2 = jnp.maximum(d2.astype(bf) + b2b_ref[...], 0)
    d3 = jnp.dot(h2, w3b_ref[...], preferred_element_type=f32)
    h3 = jnp.maximum(d3.astype(bf) + b3b_ref[...], 0)
    logit_t = (jnp.dot(wob_ref[...], h3.T, preferred_element_type=f32)
               + bo_ref[...])
    out_ref[...] = jax.nn.sigmoid(logit_t)


def _mlp(x, Pu, Eu, W1, b1, W2, b2, W3, b3, Wo, bo, blk=8192, interpret=False):
    B, D = x.shape
    H1 = W1.shape[1]
    H2 = W2.shape[1]
    H3 = W3.shape[1]
    grid = (B // blk,)
    full = lambda shape: pl.BlockSpec(shape, lambda i: tuple(0 for _ in shape))
    return pl.pallas_call(
        _mlp_body,
        grid=grid,
        in_specs=[
            pl.BlockSpec((blk, D), lambda i: (i, 0)),
            full((1, D)),
            full((1, D)),
            full((2 * D, H1)),
            full((1, H1)),
            full((H1, H2)),
            full((1, H2)),
            full((H2, H3)),
            full((1, H3)),
            full((1, H3)),
            full((1, 1)),
        ],
        out_specs=pl.BlockSpec((1, blk), lambda i: (0, i)),
        out_shape=jax.ShapeDtypeStruct((1, B), jnp.float32),
        scratch_shapes=[
            pltpu.VMEM((1, H1), jnp.bfloat16),
            pltpu.VMEM((D, H1), jnp.bfloat16),
            pltpu.VMEM((H1, H2), jnp.bfloat16),
            pltpu.VMEM((H2, H3), jnp.bfloat16),
            pltpu.VMEM((1, H3), jnp.bfloat16),
            pltpu.VMEM((1, H2), jnp.bfloat16),
            pltpu.VMEM((1, H3), jnp.bfloat16),
        ],
        compiler_params=pltpu.CompilerParams(
            dimension_semantics=("arbitrary",),
        ),
        interpret=interpret,
    )(x, Pu, Eu, W1, b1, W2, b2, W3, b3, Wo, bo)


def kernel(item_indices, Pu, Eu, Item, W1, b1, W2, b2, W3, b3, Wo, bo):
    B = item_indices.shape[0]
    V, D = Item.shape
    gather = _make_sc_gather(V, D, B)
    x = gather(item_indices.astype(jnp.int32), Item)
    out = _mlp(
        x, Pu, Eu, W1, b1.reshape(1, -1), W2, b2.reshape(1, -1),
        W3, b3.reshape(1, -1), Wo.reshape(1, -1), bo.reshape(1, 1),
    )
    return out.reshape(B, 1)


# speedup vs baseline: 1.2501x; 1.0030x over previous
import functools

import jax
import jax.numpy as jnp
from jax import lax
from jax.experimental import pallas as pl
from jax.experimental.pallas import tpu as pltpu
from jax.experimental.pallas import tpu_sc as plsc


def _make_sc_gather(V, D, B, num_cores=None):
    info = plsc.get_sparse_core_info()
    NC, NS = info.num_cores, info.num_subcores
    if num_cores is not None:
        NC = num_cores
    NW = NC * NS
    b_per_w = B // NW
    assert B % NW == 0 and b_per_w % 128 == 0
    nchunk = b_per_w // 128
    mesh = plsc.VectorSubcoreMesh(core_axis_name="c", subcore_axis_name="s",
                                  num_cores=NC)

    nbuf = min(nchunk, 4)

    def gather_k(idx_hbm, table_hbm, out_hbm, idx_v, rows_v, *sems):
        gsems = sems[:nbuf]
        wsems = sems[nbuf:]
        wid = lax.axis_index("s") * NC + lax.axis_index("c")
        base = wid * b_per_w

        pltpu.sync_copy(idx_hbm.at[wid], idx_v)

        def gather_chunk(j):
            return pltpu.async_copy(
                table_hbm.at[idx_v.at[j]],
                rows_v.at[j % nbuf],
                gsems[j % nbuf],
            )

        gathers = [gather_chunk(j) for j in range(nbuf)]
        gathers += [None] * (nchunk - nbuf)
        writes = [None] * nchunk
        for j in range(nchunk):
            slot = j % nbuf
            gathers[j].wait()
            writes[j] = pltpu.async_copy(
                rows_v.at[slot],
                out_hbm.at[pl.ds(base + j * 128, 128)],
                wsems[slot],
            )
            k = j + nbuf
            if k < nchunk:
                writes[j].wait()
                gathers[k] = gather_chunk(k)
        for j in range(nchunk - nbuf, nchunk):
            writes[j].wait()

    gather_call = functools.partial(
        pl.kernel,
        mesh=mesh,
        out_type=jax.ShapeDtypeStruct((B, D), jnp.float32),
        scratch_types=[
            pltpu.VMEM((nchunk, 128), jnp.int32),
            pltpu.VMEM((nbuf, 128, D), jnp.float32),
        ] + [pltpu.SemaphoreType.DMA] * (2 * nbuf),
    )(gather_k)

    def run(item_indices, Item):
        idx3 = item_indices.reshape(NW, nchunk, 128)
        return gather_call(idx3, Item)

    return run


def _mlp_body(x_ref, pu_ref, eu_ref, w1_ref, b1_ref, w2_ref, b2_ref,
              w3_ref, b3_ref, wo_ref, bo_ref, out_ref,
              h0_ref, w1b_ref, w2b_ref, w3b_ref, wob_ref, b2b_ref, b3b_ref):
    f32, bf = jnp.float32, jnp.bfloat16
    D = pu_ref.shape[1]

    @pl.when(pl.program_id(0) == 0)
    def _():
        u = pu_ref[...] + eu_ref[...]
        h0_ref[...] = (
            jnp.dot(u, w1_ref[0:D, :], preferred_element_type=f32)
            + b1_ref[...]).astype(bf)
        w1b_ref[...] = w1_ref[D:, :].astype(bf)
        w2b_ref[...] = w2_ref[...].astype(bf)
        w3b_ref[...] = w3_ref[...].astype(bf)
        wob_ref[...] = wo_ref[...].astype(bf)
        b2b_ref[...] = b2_ref[...].astype(bf)
        b3b_ref[...] = b3_ref[...].astype(bf)

    h0 = h0_ref[...]
    x = x_ref[...].astype(bf)
    d1 = jnp.dot(x, w1b_ref[...], preferred_element_type=f32)
    h1 = jnp.maximum(d1.astype(bf) + h0, 0)
    d2 = jnp.dot(h1, w2b_ref[...], preferred_element_type=f32)
    h2 = jnp.maximum(d2.astype(bf) + b2b_ref[...], 0)
    d3 = jnp.dot(h2, w3b_ref[...], preferred_element_type=f32)
    h3 = jnp.maximum(d3.astype(bf) + b3b_ref[...], 0)
    logit_t = (jnp.dot(wob_ref[...], h3.T, preferred_element_type=f32)
               + bo_ref[...])
    out_ref[...] = jax.nn.sigmoid(logit_t)


def _mlp(x, Pu, Eu, W1, b1, W2, b2, W3, b3, Wo, bo, blk=4096, interpret=False):
    B, D = x.shape
    H1 = W1.shape[1]
    H2 = W2.shape[1]
    H3 = W3.shape[1]
    grid = (B // blk,)
    full = lambda shape: pl.BlockSpec(shape, lambda i: tuple(0 for _ in shape))
    return pl.pallas_call(
        _mlp_body,
        grid=grid,
        in_specs=[
            pl.BlockSpec((blk, D), lambda i: (i, 0)),
            full((1, D)),
            full((1, D)),
            full((2 * D, H1)),
            full((1, H1)),
            full((H1, H2)),
            full((1, H2)),
            full((H2, H3)),
            full((1, H3)),
            full((1, H3)),
            full((1, 1)),
        ],
        out_specs=pl.BlockSpec((1, blk), lambda i: (0, i)),
        out_shape=jax.ShapeDtypeStruct((1, B), jnp.float32),
        scratch_shapes=[
            pltpu.VMEM((1, H1), jnp.bfloat16),
            pltpu.VMEM((D, H1), jnp.bfloat16),
            pltpu.VMEM((H1, H2), jnp.bfloat16),
            pltpu.VMEM((H2, H3), jnp.bfloat16),
            pltpu.VMEM((1, H3), jnp.bfloat16),
            pltpu.VMEM((1, H2), jnp.bfloat16),
            pltpu.VMEM((1, H3), jnp.bfloat16),
        ],
        compiler_params=pltpu.CompilerParams(
            dimension_semantics=("arbitrary",),
        ),
        interpret=interpret,
    )(x, Pu, Eu, W1, b1, W2, b2, W3, b3, Wo, bo)


def kernel(item_indices, Pu, Eu, Item, W1, b1, W2, b2, W3, b3, Wo, bo):
    B = item_indices.shape[0]
    V, D = Item.shape
    gather = _make_sc_gather(V, D, B)
    x = gather(item_indices.astype(jnp.int32), Item)
    out = _mlp(
        x, Pu, Eu, W1, b1.reshape(1, -1), W2, b2.reshape(1, -1),
        W3, b3.reshape(1, -1), Wo.reshape(1, -1), bo.reshape(1, 1),
    )
    return out.reshape(B, 1)
